# baseline (speedup 1.0000x reference)
import jax
import jax.numpy as jnp
from jax.experimental import pallas as pl
from jax.experimental.pallas import tpu as pltpu

_EPS = 1e-5

_ROWDOT = (((0,), (0,)), ((), ()))
_HI = jax.lax.Precision.HIGHEST


def _bf(v):
    return v.astype(jnp.bfloat16).astype(jnp.float32)


def _fused_mlp_kernel(x_ref, m_ref, W1_ref, b1_ref, g1_ref, be1_ref,
                      W2_ref, b2_ref, g2_ref, be2_ref, W3_ref, b3_ref,
                      out_ref,
                      sx_ref, Sxx_ref, sa1_ref, S_ref, cnt_ref,
                      V1_ref, t1_ref, sh1_ref, V2_ref, t2_ref, sh2_ref,
                      W3b_ref):
    phase = pl.program_id(0)
    i = pl.program_id(1)

    @pl.when((phase == 0) & (i == 0))
    def _init():
        sx_ref[...] = jnp.zeros_like(sx_ref)
        Sxx_ref[...] = jnp.zeros_like(Sxx_ref)
        sa1_ref[...] = jnp.zeros_like(sa1_ref)
        S_ref[...] = jnp.zeros_like(S_ref)
        cnt_ref[0, 0] = 0.0
        W3b_ref[...] = W3_ref[...].astype(jnp.bfloat16)

    x = x_ref[...]
    m = m_ref[...]

    @pl.when(phase == 0)
    def _p0():
        xb = x.astype(jnp.bfloat16)
        mb = m.astype(jnp.bfloat16)
        sx_ref[...] += jax.lax.dot_general(
            mb, xb, _ROWDOT, preferred_element_type=jnp.float32)
        Sxx_ref[...] += jax.lax.dot_general(
            xb * mb, xb, _ROWDOT, preferred_element_type=jnp.float32)
        cnt_ref[0, 0] += jnp.sum(m)

    @pl.when((phase == 1) & (i == 0))
    def _bn1_params():
        c = jnp.maximum(cnt_ref[0, 0], 1.0)
        W1bf = _bf(W1_ref[...])
        s1 = jnp.dot(sx_ref[...], W1bf, precision=_HI,
                     preferred_element_type=jnp.float32)
        q1 = jnp.sum(W1bf * jnp.dot(Sxx_ref[...], W1bf, precision=_HI,
                                    preferred_element_type=jnp.float32),
                     axis=0, keepdims=True)
        mean = s1 / c
        var = q1 / c - mean * mean
        sc = g1_ref[...] * jax.lax.rsqrt(var + _EPS)
        V1_ref[...] = W1_ref[...].astype(jnp.bfloat16)
        t1_ref[...] = sc
        sh1_ref[...] = be1_ref[...] - mean * sc

    @pl.when(phase >= 1)
    def _p12():
        h1 = jnp.dot(x.astype(jnp.bfloat16), V1_ref[...],
                     preferred_element_type=jnp.float32)
        a1 = jnp.maximum(h1 * t1_ref[...] + sh1_ref[...], 0.0)
        a1b = a1.astype(jnp.bfloat16)

        @pl.when(phase == 1)
        def _p1():
            mb = m.astype(jnp.bfloat16)
            sa1_ref[...] += jax.lax.dot_general(
                mb, a1b, _ROWDOT, preferred_element_type=jnp.float32)
            S_ref[...] += jax.lax.dot_general(
                a1b * mb, a1b, _ROWDOT, preferred_element_type=jnp.float32)

        @pl.when(phase == 2)
        def _p2():
            @pl.when(i == 0)
            def _bn2_params():
                c = jnp.maximum(cnt_ref[0, 0], 1.0)
                W2bf = _bf(W2_ref[...])
                s2 = jnp.dot(sa1_ref[...], W2bf, precision=_HI,
                             preferred_element_type=jnp.float32)
                q2 = jnp.sum(W2bf * jnp.dot(S_ref[...], W2bf, precision=_HI,
                                            preferred_element_type=jnp.float32),
                             axis=0, keepdims=True)
                mean = s2 / c
                var = q2 / c - mean * mean
                sc = g2_ref[...] * jax.lax.rsqrt(var + _EPS)
                V2_ref[...] = W2_ref[...].astype(jnp.bfloat16)
                t2_ref[...] = sc
                sh2_ref[...] = be2_ref[...] - mean * sc

            h2 = jnp.dot(a1b, V2_ref[...], preferred_element_type=jnp.float32)
            a2 = jnp.maximum(h2 * t2_ref[...] + sh2_ref[...], 0.0)
            y = jnp.dot(a2.astype(jnp.bfloat16), W3b_ref[...],
                        preferred_element_type=jnp.float32) + b3_ref[...]
            out_ref[...] = y * m


def _fused_mlp(x, m, W1, b1, g1, be1, W2, b2, g2, be2, W3, b3, blk):
    R, IN = x.shape
    H1 = W1.shape[1]
    H2 = W2.shape[1]
    OUTD = W3.shape[1]
    nb = R // blk

    def rows(p, i):
        return (i, 0)

    def whole(p, i):
        return (0, 0)

    out = pl.pallas_call(
        _fused_mlp_kernel,
        grid=(3, nb),
        in_specs=[
            pl.BlockSpec((blk, IN), rows),
            pl.BlockSpec((blk, 1), rows),
            pl.BlockSpec((IN, H1), whole),
            pl.BlockSpec((1, H1), whole),
            pl.BlockSpec((1, H1), whole),
            pl.BlockSpec((1, H1), whole),
            pl.BlockSpec((H1, H2), whole),
            pl.BlockSpec((1, H2), whole),
            pl.BlockSpec((1, H2), whole),
            pl.BlockSpec((1, H2), whole),
            pl.BlockSpec((H2, OUTD), whole),
            pl.BlockSpec((1, OUTD), whole),
        ],
        out_specs=pl.BlockSpec((blk, OUTD), lambda p, i: (jnp.where(p == 2, i, 0), 0)),
        out_shape=jax.ShapeDtypeStruct((R, OUTD), jnp.float32),
        scratch_shapes=[
            pltpu.VMEM((1, IN), jnp.float32),
            pltpu.VMEM((IN, IN), jnp.float32),
            pltpu.VMEM((1, H1), jnp.float32),
            pltpu.VMEM((H1, H1), jnp.float32),
            pltpu.SMEM((1, 1), jnp.float32),
            pltpu.VMEM((IN, H1), jnp.bfloat16),
            pltpu.VMEM((1, H1), jnp.float32),
            pltpu.VMEM((1, H1), jnp.float32),
            pltpu.VMEM((H1, H2), jnp.bfloat16),
            pltpu.VMEM((1, H2), jnp.float32),
            pltpu.VMEM((1, H2), jnp.float32),
            pltpu.VMEM((H2, OUTD), jnp.bfloat16),
        ],
        compiler_params=pltpu.CompilerParams(
            dimension_semantics=("arbitrary", "arbitrary"),
        ),
    )(x, m, W1, b1.reshape(1, -1), g1.reshape(1, -1), be1.reshape(1, -1),
      W2, b2.reshape(1, -1), g2.reshape(1, -1), be2.reshape(1, -1),
      W3, b3.reshape(1, -1))
    return out


def kernel(bbox_ltwh, feats_masks, W1, b1, g1, be1, W2, b2, g2, be2, W3, b3):
    B, N, T, IN = bbox_ltwh.shape
    R = B * N
    x = bbox_ltwh[:, :, 0].reshape(R, IN)
    m = feats_masks[:, :, 0].reshape(R, 1).astype(jnp.float32)
    out = _fused_mlp(x, m, W1, b1, g1, be1, W2, b2, g2, be2, W3, b3, blk=2048)
    return out.reshape(B, N, W3.shape[1])


# speedup vs baseline: 1.1532x; 1.1532x over previous
import jax
import jax.numpy as jnp
from jax.experimental import pallas as pl
from jax.experimental.pallas import tpu as pltpu

_EPS = 1e-5

_ROWDOT = (((0,), (0,)), ((), ()))
_HI = jax.lax.Precision.HIGHEST


def _bf(v):
    return v.astype(jnp.bfloat16).astype(jnp.float32)


def _fused_mlp_kernel(x_ref, m_ref, W1_ref, b1_ref, g1_ref, be1_ref,
                      W2_ref, b2_ref, g2_ref, be2_ref, W3_ref, b3_ref,
                      out_ref,
                      sx_ref, Sxx_ref, sa1_ref, S_ref, cnt_ref,
                      V1_ref, t1_ref, sh1_ref, V2_ref, t2_ref, sh2_ref,
                      W3b_ref):
    phase = pl.program_id(0)
    i = pl.program_id(1)

    @pl.when((phase == 0) & (i == 0))
    def _init():
        sx_ref[...] = jnp.zeros_like(sx_ref)
        Sxx_ref[...] = jnp.zeros_like(Sxx_ref)
        sa1_ref[...] = jnp.zeros_like(sa1_ref)
        S_ref[...] = jnp.zeros_like(S_ref)
        cnt_ref[0, 0] = 0.0
        W3b_ref[...] = W3_ref[...].astype(jnp.bfloat16)

    x = x_ref[...]
    m = m_ref[...]

    @pl.when(phase == 0)
    def _p0():
        xb = x.astype(jnp.bfloat16)
        mb = m.astype(jnp.bfloat16)
        sx_ref[...] += jax.lax.dot_general(
            mb, xb, _ROWDOT, preferred_element_type=jnp.float32)
        Sxx_ref[...] += jax.lax.dot_general(
            xb * mb, xb, _ROWDOT, preferred_element_type=jnp.float32)
        cnt_ref[0, 0] += jnp.sum(m)

    @pl.when((phase == 1) & (i == 0))
    def _bn1_params():
        c = jnp.maximum(cnt_ref[0, 0], 1.0)
        W1bf = _bf(W1_ref[...])
        s1 = jnp.dot(sx_ref[...], W1bf, precision=_HI,
                     preferred_element_type=jnp.float32)
        q1 = jnp.sum(W1bf * jnp.dot(Sxx_ref[...], W1bf, precision=_HI,
                                    preferred_element_type=jnp.float32),
                     axis=0, keepdims=True)
        mean = s1 / c
        var = q1 / c - mean * mean
        sc = g1_ref[...] * jax.lax.rsqrt(var + _EPS)
        V1_ref[...] = W1_ref[...].astype(jnp.bfloat16)
        t1_ref[...] = sc
        sh1_ref[...] = be1_ref[...] - mean * sc

    @pl.when(phase >= 1)
    def _p12():
        h1 = jnp.dot(x.astype(jnp.bfloat16), V1_ref[...],
                     preferred_element_type=jnp.float32)
        a1 = jnp.maximum(h1 * t1_ref[...] + sh1_ref[...], 0.0)
        a1b = a1.astype(jnp.bfloat16)

        @pl.when(phase == 1)
        def _p1():
            mb = m.astype(jnp.bfloat16)
            sa1_ref[...] += jax.lax.dot_general(
                mb, a1b, _ROWDOT, preferred_element_type=jnp.float32)
            S_ref[...] += jax.lax.dot_general(
                a1b * mb, a1b, _ROWDOT, preferred_element_type=jnp.float32)

        @pl.when(phase == 2)
        def _p2():
            @pl.when(i == 0)
            def _bn2_params():
                c = jnp.maximum(cnt_ref[0, 0], 1.0)
                W2bf = _bf(W2_ref[...])
                s2 = jnp.dot(sa1_ref[...], W2bf, precision=_HI,
                             preferred_element_type=jnp.float32)
                q2 = jnp.sum(W2bf * jnp.dot(S_ref[...], W2bf, precision=_HI,
                                            preferred_element_type=jnp.float32),
                             axis=0, keepdims=True)
                mean = s2 / c
                var = q2 / c - mean * mean
                sc = g2_ref[...] * jax.lax.rsqrt(var + _EPS)
                V2_ref[...] = W2_ref[...].astype(jnp.bfloat16)
                t2_ref[...] = sc
                sh2_ref[...] = be2_ref[...] - mean * sc

            h2 = jnp.dot(a1b, V2_ref[...], preferred_element_type=jnp.float32)
            a2 = jnp.maximum(h2 * t2_ref[...] + sh2_ref[...], 0.0)
            y = jnp.dot(a2.astype(jnp.bfloat16), W3b_ref[...],
                        preferred_element_type=jnp.float32) + b3_ref[...]
            out_ref[...] = y * m


def _fused_mlp(x, m, W1, b1, g1, be1, W2, b2, g2, be2, W3, b3, blk):
    R, IN = x.shape
    H1 = W1.shape[1]
    H2 = W2.shape[1]
    OUTD = W3.shape[1]
    nb = R // blk

    def rows(p, i):
        return (i, 0)

    def whole(p, i):
        return (0, 0)

    out = pl.pallas_call(
        _fused_mlp_kernel,
        grid=(3, nb),
        in_specs=[
            pl.BlockSpec((blk, IN), rows),
            pl.BlockSpec((blk, 1), rows),
            pl.BlockSpec((IN, H1), whole),
            pl.BlockSpec((1, H1), whole),
            pl.BlockSpec((1, H1), whole),
            pl.BlockSpec((1, H1), whole),
            pl.BlockSpec((H1, H2), whole),
            pl.BlockSpec((1, H2), whole),
            pl.BlockSpec((1, H2), whole),
            pl.BlockSpec((1, H2), whole),
            pl.BlockSpec((H2, OUTD), whole),
            pl.BlockSpec((1, OUTD), whole),
        ],
        out_specs=pl.BlockSpec((blk, OUTD), lambda p, i: (jnp.where(p == 2, i, 0), 0)),
        out_shape=jax.ShapeDtypeStruct((R, OUTD), jnp.float32),
        scratch_shapes=[
            pltpu.VMEM((1, IN), jnp.float32),
            pltpu.VMEM((IN, IN), jnp.float32),
            pltpu.VMEM((1, H1), jnp.float32),
            pltpu.VMEM((H1, H1), jnp.float32),
            pltpu.SMEM((1, 1), jnp.float32),
            pltpu.VMEM((IN, H1), jnp.bfloat16),
            pltpu.VMEM((1, H1), jnp.float32),
            pltpu.VMEM((1, H1), jnp.float32),
            pltpu.VMEM((H1, H2), jnp.bfloat16),
            pltpu.VMEM((1, H2), jnp.float32),
            pltpu.VMEM((1, H2), jnp.float32),
            pltpu.VMEM((H2, OUTD), jnp.bfloat16),
        ],
        compiler_params=pltpu.CompilerParams(
            dimension_semantics=("arbitrary", "arbitrary"),
        ),
    )(x, m, W1, b1.reshape(1, -1), g1.reshape(1, -1), be1.reshape(1, -1),
      W2, b2.reshape(1, -1), g2.reshape(1, -1), be2.reshape(1, -1),
      W3, b3.reshape(1, -1))
    return out


def kernel(bbox_ltwh, feats_masks, W1, b1, g1, be1, W2, b2, g2, be2, W3, b3):
    B, N, T, IN = bbox_ltwh.shape
    R = B * N
    x = bbox_ltwh[:, :, 0].reshape(R, IN)
    m = feats_masks[:, :, 0].reshape(R, 1).astype(jnp.float32)
    out = _fused_mlp(x, m, W1, b1, g1, be1, W2, b2, g2, be2, W3, b3, blk=4096)
    return out.reshape(B, N, W3.shape[1])
